# baseline (speedup 1.0000x reference)
import jax
import jax.numpy as jnp
from jax import lax
from jax.experimental import pallas as pl
from jax.experimental.pallas import tpu as pltpu
from jax.experimental.pallas import tpu_sc as plsc

N_NODES = 10000
N_EDGES = 320000
EMB = 128
NUM_EMBEDDINGS = 95

NC, NS, LANES = 2, 16, 16
NW = NC * NS

BLK = 12800
NB = N_EDGES // BLK
NVB = BLK // LANES
MAX_BPW = (NB + NW - 1) // NW


def _sc_gather_body(z_hbm, ii_hbm, jj_hbm, zi_hbm, zj_hbm,
                    z_v, ii_v, jj_v, oi_v, oj_v, sem):
    wid = lax.axis_index("s") * NC + lax.axis_index("c")
    pltpu.async_copy(z_hbm, z_v, sem).wait()

    def do_block(blk):
        base = blk * BLK
        ci = pltpu.async_copy(ii_hbm.at[pl.ds(base, BLK)], ii_v, sem)
        cj = pltpu.async_copy(jj_hbm.at[pl.ds(base, BLK)], jj_v, sem)
        ci.wait()
        cj.wait()

        def body(k, c):
            for u in range(5):
                s = pl.ds((k * 5 + u) * LANES, LANES)
                oi_v[s] = plsc.load_gather(z_v, [ii_v[s]])
                oj_v[s] = plsc.load_gather(z_v, [jj_v[s]])
            return c

        lax.fori_loop(0, NVB // 5, body, 0)
        pltpu.sync_copy(oi_v, zi_hbm.at[blk, 0])
        pltpu.sync_copy(oj_v, zj_hbm.at[blk, 0])

    for t in range(MAX_BPW):
        blk = wid + NW * t
        if (t + 1) * NW <= NB:
            do_block(blk)
        else:
            @pl.when(blk < NB)
            def _():
                do_block(blk)


def _sc_gather(Z, ii, jj):
    mesh = plsc.VectorSubcoreMesh(core_axis_name="c", subcore_axis_name="s")
    f = pl.kernel(
        _sc_gather_body,
        mesh=mesh,
        out_type=(jax.ShapeDtypeStruct((NB, 1, BLK), jnp.int32),
                  jax.ShapeDtypeStruct((NB, 1, BLK), jnp.int32)),
        scratch_types=[
            pltpu.VMEM((N_NODES,), jnp.int32),
            pltpu.VMEM((BLK,), jnp.int32),
            pltpu.VMEM((BLK,), jnp.int32),
            pltpu.VMEM((BLK,), jnp.int32),
            pltpu.VMEM((BLK,), jnp.int32),
            pltpu.SemaphoreType.DMA,
        ],
        compiler_params=pltpu.CompilerParams(needs_layout_passes=False),
    )
    return f(Z, ii, jj)


def _silu_half(t):
    return t + t * jnp.tanh(t)


def _tc_body(rbf_ref, zi_ref, zj_ref, e_ref, wd_ref, wr_ref, br_ref, bd_ref,
             out_ref, wcat_scr, cat_scr, ci_scr):
    bf = jnp.bfloat16

    @pl.when(pl.program_id(0) == 0)
    def _():
        ci_scr[...] = lax.broadcasted_iota(jnp.int16, (BLK, 128), 1)
        ew = e_ref[...]
        wcat_scr[0:128, :] = (0.5 * (jnp.dot(
            ew, wd_ref[0:128, :],
            preferred_element_type=jnp.float32) + bd_ref[...])).astype(bf)
        wcat_scr[128:256, :] = (0.5 * jnp.dot(
            ew, wd_ref[128:256, :],
            preferred_element_type=jnp.float32)).astype(bf)
        wcat_scr[256:384, :] = (0.5 * wd_ref[256:384, :]).astype(bf)

    zi = zi_ref[0, 0, :].astype(jnp.int16)
    zj = zj_ref[0, 0, :].astype(jnp.int16)
    ci = ci_scr[...]
    one = jnp.ones((), bf)
    zero = jnp.zeros((), bf)
    cat_scr[:, 0:128] = jnp.where(ci == zi[:, None], one, zero)
    cat_scr[:, 128:256] = jnp.where(ci == zj[:, None], one, zero)
    rt = lax.dot_general(rbf_ref[...].astype(bf),
                         (0.5 * wr_ref[...]).astype(bf),
                         (((0,), (0,)), ((), ())),
                         preferred_element_type=jnp.float32) + 0.5 * br_ref[...]
    cat_scr[:, 256:384] = _silu_half(rt).astype(bf)
    t = jnp.dot(cat_scr[...], wcat_scr[...],
                preferred_element_type=jnp.float32)
    out_ref[...] = _silu_half(t)


def kernel(Z, rbf, idnb_i, idnb_j, embeddings, W_rbf, b_rbf, W_dense, b_dense):
    Z = Z.astype(jnp.int32)
    ii = idnb_i.astype(jnp.int32)
    jj = idnb_j.astype(jnp.int32)

    zi3, zj3 = _sc_gather(Z, ii, jj)

    epad = jnp.pad(embeddings, ((0, 128 - NUM_EMBEDDINGS), (0, 0)))
    rbf_t = jnp.swapaxes(rbf, 0, 1)
    br = b_rbf.reshape(1, EMB)
    bd = b_dense.reshape(1, EMB)

    return pl.pallas_call(
        _tc_body,
        grid=(NB,),
        in_specs=[
            pl.BlockSpec((6, BLK), lambda i: (0, i)),
            pl.BlockSpec((1, 1, BLK), lambda i: (i, 0, 0)),
            pl.BlockSpec((1, 1, BLK), lambda i: (i, 0, 0)),
            pl.BlockSpec((128, 128), lambda i: (0, 0)),
            pl.BlockSpec((384, 128), lambda i: (0, 0)),
            pl.BlockSpec((6, 128), lambda i: (0, 0)),
            pl.BlockSpec((1, 128), lambda i: (0, 0)),
            pl.BlockSpec((1, 128), lambda i: (0, 0)),
        ],
        out_specs=pl.BlockSpec((BLK, EMB), lambda i: (i, 0)),
        out_shape=jax.ShapeDtypeStruct((N_EDGES, EMB), jnp.float32),
        scratch_shapes=[pltpu.VMEM((384, EMB), jnp.bfloat16),
                        pltpu.VMEM((BLK, 384), jnp.bfloat16),
                        pltpu.VMEM((BLK, 128), jnp.int16)],
        compiler_params=pltpu.CompilerParams(
            dimension_semantics=("arbitrary",),
            fuse_transposed_lhs_in_matmul=True),
    )(rbf_t, zi3, zj3, epad, W_dense, W_rbf, br, bd)


# speedup vs baseline: 1.0421x; 1.0421x over previous
import jax
import jax.numpy as jnp
from jax import lax
from jax.experimental import pallas as pl
from jax.experimental.pallas import tpu as pltpu
from jax.experimental.pallas import tpu_sc as plsc

N_NODES = 10000
N_EDGES = 320000
EMB = 128
NUM_EMBEDDINGS = 95

NC, NS, LANES = 2, 16, 16
NW = NC * NS

BLK = 12800
NB = N_EDGES // BLK
NVB = BLK // LANES
MAX_BPW = (NB + NW - 1) // NW


def _sc_gather_body(z_hbm, ii_hbm, jj_hbm, zi_hbm, zj_hbm,
                    z_v, ii_v, jj_v, oi_v, oj_v, sem):
    wid = lax.axis_index("s") * NC + lax.axis_index("c")
    pltpu.async_copy(z_hbm, z_v, sem).wait()

    def do_block(blk):
        base = blk * BLK
        ci = pltpu.async_copy(ii_hbm.at[pl.ds(base, BLK)], ii_v, sem)
        cj = pltpu.async_copy(jj_hbm.at[pl.ds(base, BLK)], jj_v, sem)
        ci.wait()
        cj.wait()

        def body(k, c):
            for u in range(5):
                s = pl.ds((k * 5 + u) * LANES, LANES)
                oi_v[s] = plsc.load_gather(z_v, [ii_v[s]])
                oj_v[s] = plsc.load_gather(z_v, [jj_v[s]])
            return c

        lax.fori_loop(0, NVB // 5, body, 0)
        pltpu.sync_copy(oi_v, zi_hbm.at[blk, 0])
        pltpu.sync_copy(oj_v, zj_hbm.at[blk, 0])

    for t in range(MAX_BPW):
        blk = wid + NW * t
        if (t + 1) * NW <= NB:
            do_block(blk)
        else:
            @pl.when(blk < NB)
            def _():
                do_block(blk)


def _sc_gather(Z, ii, jj):
    mesh = plsc.VectorSubcoreMesh(core_axis_name="c", subcore_axis_name="s")
    f = pl.kernel(
        _sc_gather_body,
        mesh=mesh,
        out_type=(jax.ShapeDtypeStruct((NB, 1, BLK), jnp.int32),
                  jax.ShapeDtypeStruct((NB, 1, BLK), jnp.int32)),
        scratch_types=[
            pltpu.VMEM((N_NODES,), jnp.int32),
            pltpu.VMEM((BLK,), jnp.int32),
            pltpu.VMEM((BLK,), jnp.int32),
            pltpu.VMEM((BLK,), jnp.int32),
            pltpu.VMEM((BLK,), jnp.int32),
            pltpu.SemaphoreType.DMA,
        ],
        compiler_params=pltpu.CompilerParams(needs_layout_passes=False),
    )
    return f(Z, ii, jj)


def _silu_half(t):
    return t + t * jnp.tanh(t)


def _tc_body(rbf_ref, zi_ref, zj_ref, e_ref, wd_ref, wr_ref, br_ref, bd_ref,
             out_ref, wcat_scr, cat_scr):
    bf = jnp.bfloat16

    @pl.when(pl.program_id(0) == 0)
    def _():
        ew = e_ref[...]
        wcat_scr[0:128, :] = (0.5 * (jnp.dot(
            ew, wd_ref[0:128, :],
            preferred_element_type=jnp.float32) + bd_ref[...])).astype(bf)
        wcat_scr[128:256, :] = (0.5 * jnp.dot(
            ew, wd_ref[128:256, :],
            preferred_element_type=jnp.float32)).astype(bf)
        wcat_scr[256:384, :] = (0.5 * wd_ref[256:384, :]).astype(bf)

    zi = zi_ref[0, 0, :].astype(jnp.int16)
    zj = zj_ref[0, 0, :].astype(jnp.int16)
    ci = lax.broadcasted_iota(jnp.int16, (BLK, 128), 1)
    one = jnp.ones((), bf)
    zero = jnp.zeros((), bf)
    cat_scr[:, 0:128] = jnp.where(ci == zi[:, None], one, zero)
    cat_scr[:, 128:256] = jnp.where(ci == zj[:, None], one, zero)
    rt = lax.dot_general(rbf_ref[...].astype(bf),
                         (0.5 * wr_ref[...]).astype(bf),
                         (((0,), (0,)), ((), ())),
                         preferred_element_type=jnp.float32) + 0.5 * br_ref[...]
    cat_scr[:, 256:384] = _silu_half(rt).astype(bf)
    t = jnp.dot(cat_scr[...], wcat_scr[...],
                preferred_element_type=jnp.float32)
    out_ref[...] = _silu_half(t)


def kernel(Z, rbf, idnb_i, idnb_j, embeddings, W_rbf, b_rbf, W_dense, b_dense):
    Z = Z.astype(jnp.int32)
    ii = idnb_i.astype(jnp.int32)
    jj = idnb_j.astype(jnp.int32)

    zi3, zj3 = _sc_gather(Z, ii, jj)

    epad = jnp.pad(embeddings, ((0, 128 - NUM_EMBEDDINGS), (0, 0)))
    rbf_t = jnp.swapaxes(rbf, 0, 1)
    br = b_rbf.reshape(1, EMB)
    bd = b_dense.reshape(1, EMB)

    return pl.pallas_call(
        _tc_body,
        grid=(NB,),
        in_specs=[
            pl.BlockSpec((6, BLK), lambda i: (0, i)),
            pl.BlockSpec((1, 1, BLK), lambda i: (i, 0, 0)),
            pl.BlockSpec((1, 1, BLK), lambda i: (i, 0, 0)),
            pl.BlockSpec((128, 128), lambda i: (0, 0)),
            pl.BlockSpec((384, 128), lambda i: (0, 0)),
            pl.BlockSpec((6, 128), lambda i: (0, 0)),
            pl.BlockSpec((1, 128), lambda i: (0, 0)),
            pl.BlockSpec((1, 128), lambda i: (0, 0)),
        ],
        out_specs=pl.BlockSpec((BLK, EMB), lambda i: (i, 0)),
        out_shape=jax.ShapeDtypeStruct((N_EDGES, EMB), jnp.float32),
        scratch_shapes=[pltpu.VMEM((384, EMB), jnp.bfloat16),
                        pltpu.VMEM((BLK, 384), jnp.bfloat16)],
        compiler_params=pltpu.CompilerParams(
            dimension_semantics=("arbitrary",),
            fuse_transposed_lhs_in_matmul=True),
    )(rbf_t, zi3, zj3, epad, W_dense, W_rbf, br, bd)
